# baseline (speedup 1.0000x reference)
import functools

import jax
import jax.numpy as jnp
from jax import lax
from jax.experimental import pallas as pl
from jax.experimental.pallas import tpu as pltpu
from jax.experimental.pallas import tpu_sc as plsc

N_NODES = 10000
N_EDGES = 320000
NP = 10240

NC = 2
NS = 16
NW = NC * NS
CH = 125
NCHUNKS = N_EDGES // CH
CH_PER_W = NCHUNKS // NW
RPS = NP // NS
ZCH = 128
NBUF = 4
DEGW = 16

_mesh = plsc.VectorSubcoreMesh(core_axis_name="c", subcore_axis_name="s")
_sc_params = pltpu.CompilerParams(use_tc_tiling_on_sc=False)


def _zero_fill(buf, h):
    @pl.loop(0, buf.shape[0])
    def _(r):
        @pl.loop(0, h // 16)
        def _(c):
            buf[r, pl.ds(c * 16, 16)] = jnp.zeros((16,), jnp.float32)


def _make_edge_scatter(h, y_in_spmem):
    spm_scratch = [pltpu.VMEM_SHARED((N_NODES, h), jnp.float32)] if y_in_spmem else []

    @functools.partial(
        pl.kernel,
        out_type=jax.ShapeDtypeStruct((NC, NP, h), jnp.float32),
        mesh=_mesh,
        scratch_types=[
            pltpu.VMEM((CH_PER_W, CH), jnp.int32),
            pltpu.VMEM((CH_PER_W, CH), jnp.int32),
            [pltpu.VMEM((CH, h), jnp.float32) for _ in range(NBUF)],
            pltpu.VMEM((ZCH, h), jnp.float32),
            pltpu.VMEM_SHARED((NP, h), jnp.float32),
            spm_scratch,
            [pltpu.SemaphoreType.DMA for _ in range(NBUF)],
            [pltpu.SemaphoreType.DMA for _ in range(NBUF)],
        ],
        compiler_params=_sc_params,
    )
    def k(src_hbm, dst_hbm, y_hbm, out_hbm, sidx, didx, bufs, zbuf,
          acc, y_spms, gsems, ssems):
        cid = lax.axis_index("c")
        sid = lax.axis_index("s")
        wid = cid * NS + sid

        _zero_fill(zbuf, h)

        @pl.loop(0, RPS // ZCH)
        def _(t):
            pltpu.sync_copy(zbuf, acc.at[pl.ds(sid * RPS + t * ZCH, ZCH)])

        if y_in_spmem:
            y_src = y_spms[0]
            nrow = N_NODES // NS
            pltpu.sync_copy(y_hbm.at[pl.ds(sid * nrow, nrow)],
                            y_src.at[pl.ds(sid * nrow, nrow)])
        else:
            y_src = y_hbm

        pltpu.sync_copy(src_hbm.at[pl.ds(wid * CH_PER_W, CH_PER_W)], sidx)
        pltpu.sync_copy(dst_hbm.at[pl.ds(wid * CH_PER_W, CH_PER_W)], didx)
        plsc.subcore_barrier()

        def gather(t, b):
            pltpu.make_async_copy(y_src.at[sidx.at[t]], bufs[b], gsems[b]).start()

        def gather_wait(b):
            pltpu.make_async_copy(y_src.at[sidx.at[0]], bufs[b], gsems[b]).wait()

        def scat(t, b):
            pltpu.async_copy(bufs[b], acc.at[didx.at[t]], ssems[b], add=True)

        def scat_wait(b):
            pltpu.make_async_copy(bufs[b], acc.at[didx.at[0]], ssems[b]).wait()

        for b in range(NBUF):
            gather(b, b)

        @pl.loop(0, CH_PER_W, step=NBUF)
        def _(t):
            for b in range(NBUF):
                gather_wait(b)
                scat(t + b, b)
            for b in range(NBUF):
                @pl.when(t + b + NBUF < CH_PER_W)
                def _(b=b):
                    scat_wait(b)
                    gather(t + b + NBUF, b)

        for b in range(NBUF):
            scat_wait(b)

        plsc.subcore_barrier()
        pltpu.sync_copy(acc.at[pl.ds(sid * RPS, RPS)],
                        out_hbm.at[cid, pl.ds(sid * RPS, RPS)])

    return k


_edge_scatter = {32: _make_edge_scatter(32, y_in_spmem=True),
                 64: _make_edge_scatter(64, y_in_spmem=False)}


@functools.partial(
    pl.kernel,
    out_type=jax.ShapeDtypeStruct((NC, NP, DEGW), jnp.float32),
    mesh=_mesh,
    scratch_types=[
        pltpu.VMEM((CH_PER_W, CH), jnp.int32),
        pltpu.VMEM((CH, DEGW), jnp.float32),
        pltpu.VMEM((ZCH, DEGW), jnp.float32),
        pltpu.VMEM_SHARED((NP, DEGW), jnp.float32),
        pltpu.SemaphoreType.DMA,
    ],
    compiler_params=_sc_params,
)
def _deg_kernel(dst_hbm, out_hbm, didx, obuf, zbuf, acc, dsem):
    cid = lax.axis_index("c")
    sid = lax.axis_index("s")
    wid = cid * NS + sid

    _zero_fill(zbuf, DEGW)

    @pl.loop(0, obuf.shape[0])
    def _(r):
        obuf[r, pl.ds(0, 16)] = jnp.ones((16,), jnp.float32)

    @pl.loop(0, RPS // ZCH)
    def _(t):
        pltpu.sync_copy(zbuf, acc.at[pl.ds(sid * RPS + t * ZCH, ZCH)])

    pltpu.sync_copy(dst_hbm.at[pl.ds(wid * CH_PER_W, CH_PER_W)], didx)
    plsc.subcore_barrier()

    @pl.loop(0, CH_PER_W, step=NBUF)
    def _(t):
        for b in range(NBUF):
            pltpu.async_copy(obuf, acc.at[didx.at[t + b]], dsem, add=True)
        for b in range(NBUF):
            pltpu.make_async_copy(obuf, acc.at[didx.at[0]], dsem).wait()

    plsc.subcore_barrier()
    pltpu.sync_copy(acc.at[pl.ds(sid * RPS, RPS)],
                    out_hbm.at[cid, pl.ds(sid * RPS, RPS)])


def _stage0_body(deg_ref, x_ref, w_ref, dinv_ref, y_ref):
    deg = (deg_ref[0, pl.ds(0, N_NODES), 0:1]
           + deg_ref[1, pl.ds(0, N_NODES), 0:1] + 1.0)
    dinv = lax.rsqrt(deg)
    dinv_ref[...] = dinv
    y_ref[...] = dinv * jnp.dot(x_ref[...], w_ref[...],
                                preferred_element_type=jnp.float32)


def _stage_mid_body(acc_ref, y_ref, dinv_ref, b_ref, w_ref, yout_ref):
    dinv = dinv_ref[...]
    acc = acc_ref[0, pl.ds(0, N_NODES), :] + acc_ref[1, pl.ds(0, N_NODES), :]
    h = jax.nn.relu(dinv * (acc + y_ref[...]) + b_ref[...])
    yout_ref[...] = dinv * jnp.dot(h, w_ref[...],
                                   preferred_element_type=jnp.float32)


def _stage3_body(acc_ref, y_ref, dinv_ref, b_ref, out_ref):
    acc = acc_ref[0, pl.ds(0, N_NODES), :] + acc_ref[1, pl.ds(0, N_NODES), :]
    z = dinv_ref[...] * (acc + y_ref[...]) + b_ref[...]
    m = jnp.max(z, axis=1, keepdims=True)
    e = jnp.exp(z - m)
    out_ref[...] = (z - m) - jnp.log(jnp.sum(e, axis=1, keepdims=True))


def _stage0(deg, x, w):
    return pl.pallas_call(
        _stage0_body,
        out_shape=(jax.ShapeDtypeStruct((N_NODES, 1), jnp.float32),
                   jax.ShapeDtypeStruct((N_NODES, w.shape[1]), jnp.float32)),
    )(deg, x, w)


def _stage_mid(acc, y, dinv, b, w):
    return pl.pallas_call(
        _stage_mid_body,
        out_shape=jax.ShapeDtypeStruct((N_NODES, w.shape[1]), jnp.float32),
    )(acc, y, dinv, b.reshape(1, -1), w)


def _stage3(acc, y, dinv, b):
    return pl.pallas_call(
        _stage3_body,
        out_shape=jax.ShapeDtypeStruct((N_NODES, y.shape[1]), jnp.float32),
    )(acc, y, dinv, b.reshape(1, -1))


def kernel(x, edge_index, W1, b1, W2, b2, W3, b3):
    src = edge_index[0].reshape(NCHUNKS, CH)
    dst = edge_index[1].reshape(NCHUNKS, CH)

    degp = _deg_kernel(dst)
    dinv, y1 = _stage0(degp, x, W1)

    acc1 = _edge_scatter[32](src, dst, y1)
    y2 = _stage_mid(acc1, y1, dinv, b1, W2)

    acc2 = _edge_scatter[64](src, dst, y2)
    y3 = _stage_mid(acc2, y2, dinv, b2, W3)

    acc3 = _edge_scatter[64](src, dst, y3)
    return _stage3(acc3, y3, dinv, b3)


# speedup vs baseline: 1.0515x; 1.0515x over previous
import functools

import jax
import jax.numpy as jnp
from jax import lax
from jax.experimental import pallas as pl
from jax.experimental.pallas import tpu as pltpu
from jax.experimental.pallas import tpu_sc as plsc

N_NODES = 10000
N_EDGES = 320000
NP = 10240

NC = 2
NS = 16
NW = NC * NS
CH = 125
NCHUNKS = N_EDGES // CH
CH_PER_W = NCHUNKS // NW
RPS = NP // NS
ZCH = 128
NBUF = 5
DEGW = 16

_mesh = plsc.VectorSubcoreMesh(core_axis_name="c", subcore_axis_name="s")
_sc_params = pltpu.CompilerParams(use_tc_tiling_on_sc=False)


def _zero_fill(buf, h):
    @pl.loop(0, buf.shape[0])
    def _(r):
        @pl.loop(0, h // 16)
        def _(c):
            buf[r, pl.ds(c * 16, 16)] = jnp.zeros((16,), jnp.float32)


def _make_edge_scatter(h, y_in_spmem):
    spm_scratch = [pltpu.VMEM_SHARED((N_NODES, h), jnp.float32)] if y_in_spmem else []

    @functools.partial(
        pl.kernel,
        out_type=jax.ShapeDtypeStruct((NC, NP, h), jnp.float32),
        mesh=_mesh,
        scratch_types=[
            pltpu.VMEM((CH_PER_W, CH), jnp.int32),
            pltpu.VMEM((CH_PER_W, CH), jnp.int32),
            [pltpu.VMEM((CH, h), jnp.float32) for _ in range(NBUF)],
            pltpu.VMEM((ZCH, h), jnp.float32),
            pltpu.VMEM_SHARED((NP, h), jnp.float32),
            spm_scratch,
            [pltpu.SemaphoreType.DMA for _ in range(NBUF)],
            [pltpu.SemaphoreType.DMA for _ in range(NBUF)],
        ],
        compiler_params=_sc_params,
    )
    def k(src_hbm, dst_hbm, y_hbm, out_hbm, sidx, didx, bufs, zbuf,
          acc, y_spms, gsems, ssems):
        cid = lax.axis_index("c")
        sid = lax.axis_index("s")
        wid = cid * NS + sid

        _zero_fill(zbuf, h)

        @pl.loop(0, RPS // ZCH)
        def _(t):
            pltpu.sync_copy(zbuf, acc.at[pl.ds(sid * RPS + t * ZCH, ZCH)])

        if y_in_spmem:
            y_src = y_spms[0]
            nrow = N_NODES // NS
            pltpu.sync_copy(y_hbm.at[pl.ds(sid * nrow, nrow)],
                            y_src.at[pl.ds(sid * nrow, nrow)])
        else:
            y_src = y_hbm

        pltpu.sync_copy(src_hbm.at[pl.ds(wid * CH_PER_W, CH_PER_W)], sidx)
        pltpu.sync_copy(dst_hbm.at[pl.ds(wid * CH_PER_W, CH_PER_W)], didx)
        plsc.subcore_barrier()

        def gather(t, b):
            pltpu.make_async_copy(y_src.at[sidx.at[t]], bufs[b], gsems[b]).start()

        def gather_wait(b):
            pltpu.make_async_copy(y_src.at[sidx.at[0]], bufs[b], gsems[b]).wait()

        def scat(t, b):
            pltpu.async_copy(bufs[b], acc.at[didx.at[t]], ssems[b], add=True)

        def scat_wait(b):
            pltpu.make_async_copy(bufs[b], acc.at[didx.at[0]], ssems[b]).wait()

        for b in range(NBUF):
            gather(b, b)

        @pl.loop(0, CH_PER_W, step=NBUF)
        def _(t):
            for b in range(NBUF):
                gather_wait(b)
                scat(t + b, b)
            for b in range(NBUF):
                @pl.when(t + b + NBUF < CH_PER_W)
                def _(b=b):
                    scat_wait(b)
                    gather(t + b + NBUF, b)

        for b in range(NBUF):
            scat_wait(b)

        plsc.subcore_barrier()
        pltpu.sync_copy(acc.at[pl.ds(sid * RPS, RPS)],
                        out_hbm.at[cid, pl.ds(sid * RPS, RPS)])

    return k


_edge_scatter = {32: _make_edge_scatter(32, y_in_spmem=False),
                 64: _make_edge_scatter(64, y_in_spmem=False)}


@functools.partial(
    pl.kernel,
    out_type=jax.ShapeDtypeStruct((NC, NP, DEGW), jnp.float32),
    mesh=_mesh,
    scratch_types=[
        pltpu.VMEM((CH_PER_W, CH), jnp.int32),
        pltpu.VMEM((CH, DEGW), jnp.float32),
        pltpu.VMEM((ZCH, DEGW), jnp.float32),
        pltpu.VMEM_SHARED((NP, DEGW), jnp.float32),
        pltpu.SemaphoreType.DMA,
    ],
    compiler_params=_sc_params,
)
def _deg_kernel(dst_hbm, out_hbm, didx, obuf, zbuf, acc, dsem):
    cid = lax.axis_index("c")
    sid = lax.axis_index("s")
    wid = cid * NS + sid

    _zero_fill(zbuf, DEGW)

    @pl.loop(0, obuf.shape[0])
    def _(r):
        obuf[r, pl.ds(0, 16)] = jnp.ones((16,), jnp.float32)

    @pl.loop(0, RPS // ZCH)
    def _(t):
        pltpu.sync_copy(zbuf, acc.at[pl.ds(sid * RPS + t * ZCH, ZCH)])

    pltpu.sync_copy(dst_hbm.at[pl.ds(wid * CH_PER_W, CH_PER_W)], didx)
    plsc.subcore_barrier()

    @pl.loop(0, CH_PER_W, step=NBUF)
    def _(t):
        for b in range(NBUF):
            pltpu.async_copy(obuf, acc.at[didx.at[t + b]], dsem, add=True)
        for b in range(NBUF):
            pltpu.make_async_copy(obuf, acc.at[didx.at[0]], dsem).wait()

    plsc.subcore_barrier()
    pltpu.sync_copy(acc.at[pl.ds(sid * RPS, RPS)],
                    out_hbm.at[cid, pl.ds(sid * RPS, RPS)])


def _stage0_body(deg_ref, x_ref, w_ref, dinv_ref, y_ref):
    deg = (deg_ref[0, pl.ds(0, N_NODES), 0:1]
           + deg_ref[1, pl.ds(0, N_NODES), 0:1] + 1.0)
    dinv = lax.rsqrt(deg)
    dinv_ref[...] = dinv
    y_ref[...] = dinv * jnp.dot(x_ref[...], w_ref[...],
                                preferred_element_type=jnp.float32)


def _stage_mid_body(acc_ref, y_ref, dinv_ref, b_ref, w_ref, yout_ref):
    dinv = dinv_ref[...]
    acc = acc_ref[0, pl.ds(0, N_NODES), :] + acc_ref[1, pl.ds(0, N_NODES), :]
    h = jax.nn.relu(dinv * (acc + y_ref[...]) + b_ref[...])
    yout_ref[...] = dinv * jnp.dot(h, w_ref[...],
                                   preferred_element_type=jnp.float32)


def _stage3_body(acc_ref, y_ref, dinv_ref, b_ref, out_ref):
    acc = acc_ref[0, pl.ds(0, N_NODES), :] + acc_ref[1, pl.ds(0, N_NODES), :]
    z = dinv_ref[...] * (acc + y_ref[...]) + b_ref[...]
    m = jnp.max(z, axis=1, keepdims=True)
    e = jnp.exp(z - m)
    out_ref[...] = (z - m) - jnp.log(jnp.sum(e, axis=1, keepdims=True))


def _stage0(deg, x, w):
    return pl.pallas_call(
        _stage0_body,
        out_shape=(jax.ShapeDtypeStruct((N_NODES, 1), jnp.float32),
                   jax.ShapeDtypeStruct((N_NODES, w.shape[1]), jnp.float32)),
    )(deg, x, w)


def _stage_mid(acc, y, dinv, b, w):
    return pl.pallas_call(
        _stage_mid_body,
        out_shape=jax.ShapeDtypeStruct((N_NODES, w.shape[1]), jnp.float32),
    )(acc, y, dinv, b.reshape(1, -1), w)


def _stage3(acc, y, dinv, b):
    return pl.pallas_call(
        _stage3_body,
        out_shape=jax.ShapeDtypeStruct((N_NODES, y.shape[1]), jnp.float32),
    )(acc, y, dinv, b.reshape(1, -1))


def kernel(x, edge_index, W1, b1, W2, b2, W3, b3):
    src = edge_index[0].reshape(NCHUNKS, CH)
    dst = edge_index[1].reshape(NCHUNKS, CH)

    degp = _deg_kernel(dst)
    dinv, y1 = _stage0(degp, x, W1)

    acc1 = _edge_scatter[32](src, dst, y1)
    y2 = _stage_mid(acc1, y1, dinv, b1, W2)

    acc2 = _edge_scatter[64](src, dst, y2)
    y3 = _stage_mid(acc2, y2, dinv, b2, W3)

    acc3 = _edge_scatter[64](src, dst, y3)
    return _stage3(acc3, y3, dinv, b3)


# speedup vs baseline: 1.1069x; 1.0527x over previous
import functools

import jax
import jax.numpy as jnp
from jax import lax
from jax.experimental import pallas as pl
from jax.experimental.pallas import tpu as pltpu
from jax.experimental.pallas import tpu_sc as plsc

N_NODES = 10000
N_EDGES = 320000
NP = 10240

NC = 2
NS = 16
NW = NC * NS
EPW = N_EDGES // NW
CH = 128
NFULL = EPW // CH
TAIL = EPW - NFULL * CH
RPS = NP // NS
ZCH = 128
NBUF = 6
DEGW = 16

_mesh = plsc.VectorSubcoreMesh(core_axis_name="c", subcore_axis_name="s")
_sc_params = pltpu.CompilerParams(use_tc_tiling_on_sc=False)


def _zero_fill(buf, h):
    @pl.loop(0, buf.shape[0])
    def _(r):
        @pl.loop(0, h // 16)
        def _(c):
            buf[r, pl.ds(c * 16, 16)] = jnp.zeros((16,), jnp.float32)


def _make_edge_scatter(h):
    @functools.partial(
        pl.kernel,
        out_type=jax.ShapeDtypeStruct((NC, NP, h), jnp.float32),
        mesh=_mesh,
        scratch_types=[
            pltpu.VMEM((EPW,), jnp.int32),
            pltpu.VMEM((EPW,), jnp.int32),
            [pltpu.VMEM((CH, h), jnp.float32) for _ in range(NBUF)],
            pltpu.VMEM((ZCH, h), jnp.float32),
            pltpu.VMEM_SHARED((NP, h), jnp.float32),
            [pltpu.SemaphoreType.DMA for _ in range(NBUF)],
            [pltpu.SemaphoreType.DMA for _ in range(NBUF)],
        ],
        compiler_params=_sc_params,
    )
    def k(edge_hbm, y_hbm, out_hbm, sidx, didx, bufs, zbuf, acc, gsems, ssems):
        cid = lax.axis_index("c")
        sid = lax.axis_index("s")
        wid = cid * NS + sid

        _zero_fill(zbuf, h)

        @pl.loop(0, RPS // ZCH)
        def _(t):
            pltpu.sync_copy(zbuf, acc.at[pl.ds(sid * RPS + t * ZCH, ZCH)])

        pltpu.sync_copy(edge_hbm.at[0, pl.ds(wid * EPW, EPW)], sidx)
        pltpu.sync_copy(edge_hbm.at[1, pl.ds(wid * EPW, EPW)], didx)
        plsc.subcore_barrier()

        def gather(t, b):
            pltpu.make_async_copy(y_hbm.at[sidx.at[pl.ds(t * CH, CH)]],
                                  bufs[b], gsems[b]).start()

        def gather_wait(b):
            pltpu.make_async_copy(y_hbm.at[sidx.at[pl.ds(0, CH)]],
                                  bufs[b], gsems[b]).wait()

        def scat(t, b):
            pltpu.async_copy(bufs[b], acc.at[didx.at[pl.ds(t * CH, CH)]],
                             ssems[b], add=True)

        def scat_wait(b):
            pltpu.make_async_copy(bufs[b], acc.at[didx.at[pl.ds(0, CH)]],
                                  ssems[b]).wait()

        for b in range(NBUF):
            gather(b, b)

        @pl.loop(0, NFULL, step=NBUF)
        def _(t):
            for b in range(NBUF):
                gather_wait(b)
                scat(t + b, b)
            for b in range(NBUF):
                @pl.when(t + b + NBUF < NFULL)
                def _(b=b):
                    scat_wait(b)
                    gather(t + b + NBUF, b)

        for b in range(NBUF):
            scat_wait(b)

        pltpu.sync_copy(y_hbm.at[sidx.at[pl.ds(NFULL * CH, TAIL)]],
                        bufs[0].at[pl.ds(0, TAIL)])
        pltpu.sync_copy(bufs[0].at[pl.ds(0, TAIL)],
                        acc.at[didx.at[pl.ds(NFULL * CH, TAIL)]], add=True)

        plsc.subcore_barrier()
        pltpu.sync_copy(acc.at[pl.ds(sid * RPS, RPS)],
                        out_hbm.at[cid, pl.ds(sid * RPS, RPS)])

    return k


_edge_scatter = {32: _make_edge_scatter(32), 64: _make_edge_scatter(64)}


@functools.partial(
    pl.kernel,
    out_type=jax.ShapeDtypeStruct((NC, NP, DEGW), jnp.float32),
    mesh=_mesh,
    scratch_types=[
        pltpu.VMEM((EPW,), jnp.int32),
        pltpu.VMEM((CH, DEGW), jnp.float32),
        pltpu.VMEM((ZCH, DEGW), jnp.float32),
        pltpu.VMEM_SHARED((NP, DEGW), jnp.float32),
        pltpu.SemaphoreType.DMA,
    ],
    compiler_params=_sc_params,
)
def _deg_kernel(edge_hbm, out_hbm, didx, obuf, zbuf, acc, dsem):
    cid = lax.axis_index("c")
    sid = lax.axis_index("s")
    wid = cid * NS + sid

    _zero_fill(zbuf, DEGW)

    @pl.loop(0, obuf.shape[0])
    def _(r):
        obuf[r, pl.ds(0, 16)] = jnp.ones((16,), jnp.float32)

    @pl.loop(0, RPS // ZCH)
    def _(t):
        pltpu.sync_copy(zbuf, acc.at[pl.ds(sid * RPS + t * ZCH, ZCH)])

    pltpu.sync_copy(edge_hbm.at[1, pl.ds(wid * EPW, EPW)], didx)
    plsc.subcore_barrier()

    @pl.loop(0, NFULL, step=NBUF)
    def _(t):
        for b in range(NBUF):
            pltpu.async_copy(obuf, acc.at[didx.at[pl.ds((t + b) * CH, CH)]],
                             dsem, add=True)
        for b in range(NBUF):
            pltpu.make_async_copy(obuf, acc.at[didx.at[pl.ds(0, CH)]],
                                  dsem).wait()

    pltpu.sync_copy(obuf.at[pl.ds(0, TAIL)],
                    acc.at[didx.at[pl.ds(NFULL * CH, TAIL)]], add=True)

    plsc.subcore_barrier()
    pltpu.sync_copy(acc.at[pl.ds(sid * RPS, RPS)],
                    out_hbm.at[cid, pl.ds(sid * RPS, RPS)])


def _stage0_body(deg_ref, x_ref, w_ref, dinv_ref, y_ref):
    deg = (deg_ref[0, pl.ds(0, N_NODES), 0:1]
           + deg_ref[1, pl.ds(0, N_NODES), 0:1] + 1.0)
    dinv = lax.rsqrt(deg)
    dinv_ref[...] = dinv
    y_ref[...] = dinv * jnp.dot(x_ref[...], w_ref[...],
                                preferred_element_type=jnp.float32)


def _stage_mid_body(acc_ref, y_ref, dinv_ref, b_ref, w_ref, yout_ref):
    dinv = dinv_ref[...]
    acc = acc_ref[0, pl.ds(0, N_NODES), :] + acc_ref[1, pl.ds(0, N_NODES), :]
    h = jax.nn.relu(dinv * (acc + y_ref[...]) + b_ref[...])
    yout_ref[...] = dinv * jnp.dot(h, w_ref[...],
                                   preferred_element_type=jnp.float32)


def _stage3_body(acc_ref, y_ref, dinv_ref, b_ref, out_ref):
    acc = acc_ref[0, pl.ds(0, N_NODES), :] + acc_ref[1, pl.ds(0, N_NODES), :]
    z = dinv_ref[...] * (acc + y_ref[...]) + b_ref[...]
    m = jnp.max(z, axis=1, keepdims=True)
    e = jnp.exp(z - m)
    out_ref[...] = (z - m) - jnp.log(jnp.sum(e, axis=1, keepdims=True))


def _stage0(deg, x, w):
    return pl.pallas_call(
        _stage0_body,
        out_shape=(jax.ShapeDtypeStruct((N_NODES, 1), jnp.float32),
                   jax.ShapeDtypeStruct((N_NODES, w.shape[1]), jnp.float32)),
    )(deg, x, w)


def _stage_mid(acc, y, dinv, b, w):
    return pl.pallas_call(
        _stage_mid_body,
        out_shape=jax.ShapeDtypeStruct((N_NODES, w.shape[1]), jnp.float32),
    )(acc, y, dinv, b.reshape(1, -1), w)


def _stage3(acc, y, dinv, b):
    return pl.pallas_call(
        _stage3_body,
        out_shape=jax.ShapeDtypeStruct((N_NODES, y.shape[1]), jnp.float32),
    )(acc, y, dinv, b.reshape(1, -1))


def kernel(x, edge_index, W1, b1, W2, b2, W3, b3):
    degp = _deg_kernel(edge_index)
    dinv, y1 = _stage0(degp, x, W1)

    acc1 = _edge_scatter[32](edge_index, y1)
    y2 = _stage_mid(acc1, y1, dinv, b1, W2)

    acc2 = _edge_scatter[64](edge_index, y2)
    y3 = _stage_mid(acc2, y2, dinv, b2, W3)

    acc3 = _edge_scatter[64](edge_index, y3)
    return _stage3(acc3, y3, dinv, b3)


# speedup vs baseline: 1.1070x; 1.0000x over previous
import functools

import jax
import jax.numpy as jnp
from jax import lax
from jax.experimental import pallas as pl
from jax.experimental.pallas import tpu as pltpu
from jax.experimental.pallas import tpu_sc as plsc

N_NODES = 10000
N_EDGES = 320000
NP = 10240

NC = 2
NS = 16
NW = NC * NS
EPW = N_EDGES // NW
CH = 128
NFULL = EPW // CH
TAIL = EPW - NFULL * CH
RPS = NP // NS
ZCH = 128
NBUF = 6
DEGW = 16

_mesh = plsc.VectorSubcoreMesh(core_axis_name="c", subcore_axis_name="s")
_sc_params = pltpu.CompilerParams(use_tc_tiling_on_sc=False)


def _zero_fill(buf, h):
    @pl.loop(0, buf.shape[0])
    def _(r):
        @pl.loop(0, h // 16)
        def _(c):
            buf[r, pl.ds(c * 16, 16)] = jnp.zeros((16,), jnp.float32)


def _make_edge_scatter(h):
    @functools.partial(
        pl.kernel,
        out_type=jax.ShapeDtypeStruct((NC, NP, h), jnp.float32),
        mesh=_mesh,
        scratch_types=[
            pltpu.VMEM((EPW,), jnp.int32),
            pltpu.VMEM((EPW,), jnp.int32),
            [pltpu.VMEM((CH, h), jnp.float32) for _ in range(NBUF)],
            pltpu.VMEM((ZCH, h), jnp.float32),
            pltpu.VMEM_SHARED((NP, h), jnp.float32),
            [pltpu.SemaphoreType.DMA for _ in range(NBUF)],
            [pltpu.SemaphoreType.DMA for _ in range(NBUF)],
        ],
        compiler_params=_sc_params,
    )
    def k(edge_hbm, y_hbm, out_hbm, sidx, didx, bufs, zbuf, acc, gsems, ssems):
        cid = lax.axis_index("c")
        sid = lax.axis_index("s")
        wid = cid * NS + sid

        _zero_fill(zbuf, h)

        @pl.loop(0, RPS // ZCH)
        def _(t):
            pltpu.sync_copy(zbuf, acc.at[pl.ds(sid * RPS + t * ZCH, ZCH)])

        pltpu.sync_copy(edge_hbm.at[0, pl.ds(wid * EPW, EPW)], sidx)
        pltpu.sync_copy(edge_hbm.at[1, pl.ds(wid * EPW, EPW)], didx)
        plsc.subcore_barrier()

        def gather(t, b):
            pltpu.make_async_copy(y_hbm.at[sidx.at[pl.ds(t * CH, CH)]],
                                  bufs[b], gsems[b]).start()

        def gather_wait(b):
            pltpu.make_async_copy(y_hbm.at[sidx.at[pl.ds(0, CH)]],
                                  bufs[b], gsems[b]).wait()

        def scat(t, b):
            pltpu.async_copy(bufs[b], acc.at[didx.at[pl.ds(t * CH, CH)]],
                             ssems[b], add=True)

        def scat_wait(b):
            pltpu.make_async_copy(bufs[b], acc.at[didx.at[pl.ds(0, CH)]],
                                  ssems[b]).wait()

        for b in range(NBUF):
            gather(b, b)

        @pl.loop(0, NFULL, step=NBUF)
        def _(t):
            for b in range(NBUF):
                gather_wait(b)
                scat(t + b, b)
            for b in range(NBUF):
                @pl.when(t + b + NBUF < NFULL)
                def _(b=b):
                    scat_wait(b)
                    gather(t + b + NBUF, b)

        for b in range(NBUF):
            scat_wait(b)

        pltpu.sync_copy(y_hbm.at[sidx.at[pl.ds(NFULL * CH, TAIL)]],
                        bufs[0].at[pl.ds(0, TAIL)])
        pltpu.sync_copy(bufs[0].at[pl.ds(0, TAIL)],
                        acc.at[didx.at[pl.ds(NFULL * CH, TAIL)]], add=True)

        plsc.subcore_barrier()
        pltpu.sync_copy(acc.at[pl.ds(sid * RPS, RPS)],
                        out_hbm.at[cid, pl.ds(sid * RPS, RPS)])

    return k


_edge_scatter = {32: _make_edge_scatter(32), 64: _make_edge_scatter(64)}


@functools.partial(
    pl.kernel,
    out_type=jax.ShapeDtypeStruct((NC, NP, DEGW), jnp.float32),
    mesh=_mesh,
    scratch_types=[
        pltpu.VMEM((EPW,), jnp.int32),
        pltpu.VMEM((CH, DEGW), jnp.float32),
        pltpu.VMEM((ZCH, DEGW), jnp.float32),
        pltpu.VMEM_SHARED((NP, DEGW), jnp.float32),
        pltpu.SemaphoreType.DMA,
    ],
    compiler_params=_sc_params,
)
def _deg_kernel(edge_hbm, out_hbm, didx, obuf, zbuf, acc, dsem):
    cid = lax.axis_index("c")
    sid = lax.axis_index("s")
    wid = cid * NS + sid

    _zero_fill(zbuf, DEGW)

    @pl.loop(0, obuf.shape[0])
    def _(r):
        obuf[r, pl.ds(0, 16)] = jnp.ones((16,), jnp.float32)

    @pl.loop(0, RPS // ZCH)
    def _(t):
        pltpu.sync_copy(zbuf, acc.at[pl.ds(sid * RPS + t * ZCH, ZCH)])

    pltpu.sync_copy(edge_hbm.at[1, pl.ds(wid * EPW, EPW)], didx)
    plsc.subcore_barrier()

    @pl.loop(0, NFULL, step=NBUF)
    def _(t):
        for b in range(NBUF):
            pltpu.async_copy(obuf, acc.at[didx.at[pl.ds((t + b) * CH, CH)]],
                             dsem, add=True)
        for b in range(NBUF):
            pltpu.make_async_copy(obuf, acc.at[didx.at[pl.ds(0, CH)]],
                                  dsem).wait()

    pltpu.sync_copy(obuf.at[pl.ds(0, TAIL)],
                    acc.at[didx.at[pl.ds(NFULL * CH, TAIL)]], add=True)

    plsc.subcore_barrier()
    pltpu.sync_copy(acc.at[pl.ds(sid * RPS, RPS)],
                    out_hbm.at[cid, pl.ds(sid * RPS, RPS)])


def _mm_body(x_ref, w_ref, xw_ref):
    xw_ref[...] = jnp.dot(x_ref[...], w_ref[...],
                          preferred_element_type=jnp.float32)


def _scale0_body(deg_ref, xw_ref, dinv_ref, y_ref):
    deg = (deg_ref[0, pl.ds(0, N_NODES), 0:1]
           + deg_ref[1, pl.ds(0, N_NODES), 0:1] + 1.0)
    dinv = lax.rsqrt(deg)
    dinv_ref[...] = dinv
    y_ref[...] = dinv * xw_ref[...]


def _stage_mid_body(acc_ref, y_ref, dinv_ref, b_ref, w_ref, yout_ref):
    dinv = dinv_ref[...]
    acc = acc_ref[0, pl.ds(0, N_NODES), :] + acc_ref[1, pl.ds(0, N_NODES), :]
    h = jax.nn.relu(dinv * (acc + y_ref[...]) + b_ref[...])
    yout_ref[...] = dinv * jnp.dot(h, w_ref[...],
                                   preferred_element_type=jnp.float32)


def _stage3_body(acc_ref, y_ref, dinv_ref, b_ref, out_ref):
    acc = acc_ref[0, pl.ds(0, N_NODES), :] + acc_ref[1, pl.ds(0, N_NODES), :]
    z = dinv_ref[...] * (acc + y_ref[...]) + b_ref[...]
    m = jnp.max(z, axis=1, keepdims=True)
    e = jnp.exp(z - m)
    out_ref[...] = (z - m) - jnp.log(jnp.sum(e, axis=1, keepdims=True))


def _mm(x, w):
    return pl.pallas_call(
        _mm_body,
        out_shape=jax.ShapeDtypeStruct((N_NODES, w.shape[1]), jnp.float32),
    )(x, w)


def _scale0(deg, xw):
    return pl.pallas_call(
        _scale0_body,
        out_shape=(jax.ShapeDtypeStruct((N_NODES, 1), jnp.float32),
                   jax.ShapeDtypeStruct((N_NODES, xw.shape[1]), jnp.float32)),
    )(deg, xw)


def _stage_mid(acc, y, dinv, b, w):
    return pl.pallas_call(
        _stage_mid_body,
        out_shape=jax.ShapeDtypeStruct((N_NODES, w.shape[1]), jnp.float32),
    )(acc, y, dinv, b.reshape(1, -1), w)


def _stage3(acc, y, dinv, b):
    return pl.pallas_call(
        _stage3_body,
        out_shape=jax.ShapeDtypeStruct((N_NODES, y.shape[1]), jnp.float32),
    )(acc, y, dinv, b.reshape(1, -1))


def kernel(x, edge_index, W1, b1, W2, b2, W3, b3):
    degp = _deg_kernel(edge_index)
    xw1 = _mm(x, W1)
    dinv, y1 = _scale0(degp, xw1)

    acc1 = _edge_scatter[32](edge_index, y1)
    y2 = _stage_mid(acc1, y1, dinv, b1, W2)

    acc2 = _edge_scatter[64](edge_index, y2)
    y3 = _stage_mid(acc2, y2, dinv, b2, W3)

    acc3 = _edge_scatter[64](edge_index, y3)
    return _stage3(acc3, y3, dinv, b3)
